# baseline (speedup 1.0000x reference)
import functools

import jax
import jax.numpy as jnp
from jax import lax
from jax.experimental import pallas as pl
from jax.experimental.pallas import tpu as pltpu
from jax.experimental.pallas import tpu_sc as plsc

N = 10000
E = 320000
F_IN = 128
HID = 128
C_OUT = 64

NPAD = 10240
NW = 32
CW = 128
NCHUNK = 80
EPT = NCHUNK * CW
EPAD = NW * EPT
RPT = NPAD // 16
BLK = 512
GRID = NPAD // BLK
NRING = 8
LAG = 7

_mesh = plsc.VectorSubcoreMesh(core_axis_name="c", subcore_axis_name="s")


@functools.partial(
    pl.kernel,
    out_type=jax.ShapeDtypeStruct((NW, NPAD), jnp.float32),
    mesh=_mesh,
    scratch_types=[
        pltpu.VMEM((EPT,), jnp.int32),
        pltpu.VMEM((NPAD,), jnp.float32),
        pltpu.VMEM((NPAD,), jnp.float32),
    ],
    compiler_params=pltpu.CompilerParams(needs_layout_passes=False),
)
def _deg_kernel(dst_hbm, out_hbm, dst_v, deg_v, deg2_v):
    c = lax.axis_index("c")
    s = lax.axis_index("s")
    wid = s * 2 + c
    pltpu.sync_copy(dst_hbm.at[wid], dst_v)
    zeros = jnp.zeros((16,), jnp.float32)
    ones = jnp.ones((16,), jnp.float32)

    def zero_body(i, _):
        for u in range(2):
            deg_v[pl.ds((i * 2 + u) * 16, 16)] = zeros
            deg2_v[pl.ds((i * 2 + u) * 16, 16)] = zeros
        return 0

    lax.fori_loop(0, NPAD // 32, zero_body, 0)

    def body(i, _):
        idx = dst_v[pl.ds(i * 32, 16)]
        idx2 = dst_v[pl.ds(i * 32 + 16, 16)]
        plsc.addupdate_scatter(deg_v, [idx], ones)
        plsc.addupdate_scatter(deg2_v, [idx2], ones)
        return 0

    lax.fori_loop(0, EPT // 32, body, 0)

    def merge(i, _):
        deg_v[pl.ds(i * 16, 16)] = (deg_v[pl.ds(i * 16, 16)] +
                                    deg2_v[pl.ds(i * 16, 16)])
        return 0

    lax.fori_loop(0, NPAD // 16, merge, 0)
    pltpu.sync_copy(deg_v, out_hbm.at[wid])


def _make_spmm(d):
    @functools.partial(
        pl.kernel,
        out_type=jax.ShapeDtypeStruct((2, NPAD, d), jnp.bfloat16),
        mesh=_mesh,
        scratch_types=[
            pltpu.VMEM((EPT,), jnp.int32),
            pltpu.VMEM((NCHUNK, CW), jnp.int32),
            [pltpu.VMEM((CW, d), jnp.bfloat16) for _ in range(NRING)],
            pltpu.VMEM_SHARED((NPAD, d), jnp.bfloat16),
            [pltpu.SemaphoreType.DMA for _ in range(NRING)],
            [pltpu.SemaphoreType.DMA for _ in range(NRING)],
        ],
        compiler_params=pltpu.CompilerParams(use_tc_tiling_on_sc=False),
    )
    def spmm(y_hbm, src_hbm, dst_hbm, zeros_hbm, out_hbm,
             src_v, dst_v, bufs, acc_sh, gsems, ssems):
        c = lax.axis_index("c")
        s = lax.axis_index("s")
        wid = s * 2 + c
        pltpu.sync_copy(zeros_hbm.at[pl.ds(s * RPT, RPT)],
                        acc_sh.at[pl.ds(s * RPT, RPT)])
        pltpu.sync_copy(src_hbm.at[wid], src_v)
        pltpu.sync_copy(dst_hbm.at[wid], dst_v)

        def src_at(j):
            return y_hbm.at[src_v.at[pl.ds(j * CW, CW)]]

        def gather(j, b):
            pltpu.async_copy(src_at(j), bufs[b], gsems[b])

        def wait_gather(j, b):
            pltpu.make_async_copy(src_at(j), bufs[b], gsems[b]).wait()

        def scatter(j, b):
            pltpu.async_copy(bufs[b], acc_sh.at[dst_v.at[j]], ssems[b],
                             add=True)

        def wait_scatter(j, b):
            pltpu.make_async_copy(bufs[b], acc_sh.at[dst_v.at[j]],
                                  ssems[b]).wait()

        plsc.subcore_barrier()
        for b in range(LAG):
            gather(b, b)

        for jj in range(NRING):
            b = jj % NRING
            wait_gather(jj, b)
            scatter(jj, b)
            bg = (jj + LAG) % NRING
            if jj + LAG >= NRING:
                wait_scatter(jj + LAG - NRING, bg)
            gather(jj + LAG, bg)

        def body(i, _):
            for b in range(NRING):
                jj = i * NRING + b
                wait_gather(jj, b)
                scatter(jj, b)
                bg = (b + LAG) % NRING
                wait_scatter(jj + LAG - NRING, bg)
                gather(jj + LAG, bg)
            return 0

        lax.fori_loop(1, NCHUNK // NRING - 1, body, 0)

        for b in range(NRING):
            jj = NCHUNK - NRING + b
            wait_gather(jj, b)
            scatter(jj, b)
            if jj + LAG < NCHUNK:
                bg = (b + LAG) % NRING
                wait_scatter(jj + LAG - NRING, bg)
                gather(jj + LAG, bg)
        for b in range(NRING):
            wait_scatter(NCHUNK - NRING + b, b)

        plsc.subcore_barrier()
        pltpu.sync_copy(acc_sh.at[pl.ds(s * RPT, RPT)],
                        out_hbm.at[c, pl.ds(s * RPT, RPT)])

    return spmm


_spmm_hid = _make_spmm(HID)
_spmm_out = _make_spmm(C_OUT)


def _dinv_body(deg_ref, dinv_ref):
    i = pl.program_id(0)
    deg = jnp.sum(deg_ref[...], axis=0)
    rowid = lax.broadcasted_iota(jnp.int32, (BLK,), 0) + i * BLK
    dinv_ref[...] = jnp.where(rowid < N, lax.rsqrt(deg + 1.0), 0.0)


def _y1_body(dinv_ref, x_ref, w1_ref, y_ref):
    xw = jnp.dot(x_ref[...], w1_ref[...], preferred_element_type=jnp.float32)
    y_ref[...] = (xw * dinv_ref[...][:, None]).astype(jnp.bfloat16)


def _y2_body(dinv_ref, s1a_ref, s1b_ref, y1_ref, b1_ref, w2_ref, y2_ref):
    dinv = dinv_ref[...]
    agg = (s1a_ref[0].astype(jnp.float32) + s1b_ref[0].astype(jnp.float32)
           + y1_ref[...].astype(jnp.float32))
    h = jnp.maximum(agg * dinv[:, None] + b1_ref[...], 0.0)
    xw2 = jnp.dot(h, w2_ref[...], preferred_element_type=jnp.float32)
    y2_ref[...] = (xw2 * dinv[:, None]).astype(jnp.bfloat16)


def _out_body(dinv_ref, s2a_ref, s2b_ref, y2_ref, b2_ref, out_ref):
    dinv = dinv_ref[...]
    agg = (s2a_ref[0].astype(jnp.float32) + s2b_ref[0].astype(jnp.float32)
           + y2_ref[...].astype(jnp.float32))
    out_ref[...] = agg * dinv[:, None] + b2_ref[...]


def _deg_spec():
    return pl.BlockSpec((NW, BLK), lambda i: (0, i))


def _dinv_spec():
    return pl.BlockSpec((BLK,), lambda i: (i,))


def _rows(d):
    return pl.BlockSpec((BLK, d), lambda i: (i, 0))


def _part_a(d):
    return pl.BlockSpec((1, BLK, d), lambda i: (0, i, 0))


def _part_b(d):
    return pl.BlockSpec((1, BLK, d), lambda i: (1, i, 0))


def _full(shape):
    return pl.BlockSpec(shape, lambda i: (0,) * len(shape))


def kernel(x, edge_index, W1, b1, W2, b2):
    src = edge_index[0]
    dst = edge_index[1]
    pad_idx = N + (jnp.arange(EPAD - E, dtype=jnp.int32) % (NPAD - N))
    src_p = jnp.concatenate([src, pad_idx]).reshape(NW, EPT)
    dst_p = jnp.concatenate([dst, pad_idx]).reshape(NW, NCHUNK, CW)
    zeros_hid = jnp.zeros((NPAD, HID), jnp.bfloat16)
    zeros_out = jnp.zeros((NPAD, C_OUT), jnp.bfloat16)

    x_p = jnp.concatenate(
        [x, jnp.zeros((NPAD - N, F_IN), jnp.float32)], axis=0)

    deg_parts = _deg_kernel(dst_p.reshape(NW, EPT))

    dinv = pl.pallas_call(
        _dinv_body,
        grid=(GRID,),
        in_specs=[_deg_spec()],
        out_specs=_dinv_spec(),
        out_shape=jax.ShapeDtypeStruct((NPAD,), jnp.float32),
    )(deg_parts)

    y1 = pl.pallas_call(
        _y1_body,
        grid=(GRID,),
        in_specs=[_dinv_spec(), _rows(F_IN), _full((F_IN, HID))],
        out_specs=_rows(HID),
        out_shape=jax.ShapeDtypeStruct((NPAD, HID), jnp.bfloat16),
    )(dinv, x_p, W1)

    s1 = _spmm_hid(y1, src_p, dst_p, zeros_hid)

    y2 = pl.pallas_call(
        _y2_body,
        grid=(GRID,),
        in_specs=[_dinv_spec(), _part_a(HID), _part_b(HID), _rows(HID),
                  _full((1, HID)), _full((HID, C_OUT))],
        out_specs=_rows(C_OUT),
        out_shape=jax.ShapeDtypeStruct((NPAD, C_OUT), jnp.bfloat16),
    )(dinv, s1, s1, y1, b1.reshape(1, HID), W2)

    s2 = _spmm_out(y2, src_p, dst_p, zeros_out)

    out = pl.pallas_call(
        _out_body,
        grid=(GRID,),
        in_specs=[_dinv_spec(), _part_a(C_OUT), _part_b(C_OUT), _rows(C_OUT),
                  _full((1, C_OUT))],
        out_specs=_rows(C_OUT),
        out_shape=jax.ShapeDtypeStruct((NPAD, C_OUT), jnp.float32),
    )(dinv, s2, s2, y2, b2.reshape(1, C_OUT))

    return out[:N]


# speedup vs baseline: 1.0452x; 1.0452x over previous
import functools

import jax
import jax.numpy as jnp
from jax import lax
from jax.experimental import pallas as pl
from jax.experimental.pallas import tpu as pltpu
from jax.experimental.pallas import tpu_sc as plsc

N = 10000
E = 320000
F_IN = 128
HID = 128
C_OUT = 64

NPAD = 10240
NW = 32
CW = 128
NCHUNK = 80
EPT = NCHUNK * CW
EPAD = NW * EPT
RPT = NPAD // 16
BLK = 512
GRID = NPAD // BLK
NRING = 8
LAG = 7

_mesh = plsc.VectorSubcoreMesh(core_axis_name="c", subcore_axis_name="s")


@functools.partial(
    pl.kernel,
    out_type=jax.ShapeDtypeStruct((NW, NPAD), jnp.float32),
    mesh=_mesh,
    scratch_types=[
        pltpu.VMEM((EPT,), jnp.int32),
        pltpu.VMEM((NPAD,), jnp.float32),
        pltpu.VMEM((NPAD,), jnp.float32),
    ],
    compiler_params=pltpu.CompilerParams(needs_layout_passes=False),
)
def _deg_kernel(dst_hbm, out_hbm, dst_v, deg_v, deg2_v):
    c = lax.axis_index("c")
    s = lax.axis_index("s")
    wid = s * 2 + c
    pltpu.sync_copy(dst_hbm.at[wid], dst_v)
    zeros = jnp.zeros((16,), jnp.float32)
    ones = jnp.ones((16,), jnp.float32)

    def zero_body(i, _):
        for u in range(2):
            deg_v[pl.ds((i * 2 + u) * 16, 16)] = zeros
            deg2_v[pl.ds((i * 2 + u) * 16, 16)] = zeros
        return 0

    lax.fori_loop(0, NPAD // 32, zero_body, 0)

    def body(i, _):
        idx = dst_v[pl.ds(i * 32, 16)]
        idx2 = dst_v[pl.ds(i * 32 + 16, 16)]
        plsc.addupdate_scatter(deg_v, [idx], ones)
        plsc.addupdate_scatter(deg2_v, [idx2], ones)
        return 0

    lax.fori_loop(0, EPT // 32, body, 0)

    def merge(i, _):
        deg_v[pl.ds(i * 16, 16)] = (deg_v[pl.ds(i * 16, 16)] +
                                    deg2_v[pl.ds(i * 16, 16)])
        return 0

    lax.fori_loop(0, NPAD // 16, merge, 0)
    pltpu.sync_copy(deg_v, out_hbm.at[wid])


def _make_spmm(d):
    @functools.partial(
        pl.kernel,
        out_type=jax.ShapeDtypeStruct((2, NPAD, d), jnp.bfloat16),
        mesh=_mesh,
        scratch_types=[
            pltpu.VMEM((EPT,), jnp.int32),
            pltpu.VMEM((NCHUNK, CW), jnp.int32),
            [pltpu.VMEM((CW, d), jnp.bfloat16) for _ in range(NRING)],
            pltpu.VMEM_SHARED((NPAD, d), jnp.bfloat16),
            [pltpu.SemaphoreType.DMA for _ in range(NRING)],
            [pltpu.SemaphoreType.DMA for _ in range(NRING)],
        ],
        compiler_params=pltpu.CompilerParams(use_tc_tiling_on_sc=False),
    )
    def spmm(y_hbm, src_hbm, dst_hbm, zeros_hbm, out_hbm,
             src_v, dst_v, bufs, acc_sh, gsems, ssems):
        c = lax.axis_index("c")
        s = lax.axis_index("s")
        wid = s * 2 + c
        pltpu.sync_copy(zeros_hbm.at[pl.ds(s * RPT, RPT)],
                        acc_sh.at[pl.ds(s * RPT, RPT)])
        pltpu.sync_copy(src_hbm.at[wid], src_v)
        pltpu.sync_copy(dst_hbm.at[wid], dst_v)

        def src_at(j):
            return y_hbm.at[src_v.at[pl.ds(j * CW, CW)]]

        def gather(j, b):
            pltpu.async_copy(src_at(j), bufs[b], gsems[b])

        def wait_gather(j, b):
            pltpu.make_async_copy(src_at(j), bufs[b], gsems[b]).wait()

        def scatter(j, b):
            pltpu.async_copy(bufs[b], acc_sh.at[dst_v.at[j]], ssems[b],
                             add=True)

        def wait_scatter(j, b):
            pltpu.make_async_copy(bufs[b], acc_sh.at[dst_v.at[j]],
                                  ssems[b]).wait()

        plsc.subcore_barrier()
        for b in range(LAG):
            gather(b, b)

        for jj in range(NRING):
            b = jj % NRING
            wait_gather(jj, b)
            scatter(jj, b)
            bg = (jj + LAG) % NRING
            if jj + LAG >= NRING:
                wait_scatter(jj + LAG - NRING, bg)
            gather(jj + LAG, bg)

        def body(i, _):
            for b in range(NRING):
                jj = i * NRING + b
                wait_gather(jj, b)
                scatter(jj, b)
                bg = (b + LAG) % NRING
                wait_scatter(jj + LAG - NRING, bg)
                gather(jj + LAG, bg)
            return 0

        lax.fori_loop(1, NCHUNK // NRING - 1, body, 0)

        for b in range(NRING):
            jj = NCHUNK - NRING + b
            wait_gather(jj, b)
            scatter(jj, b)
            if jj + LAG < NCHUNK:
                bg = (b + LAG) % NRING
                wait_scatter(jj + LAG - NRING, bg)
                gather(jj + LAG, bg)
        for b in range(NRING):
            wait_scatter(NCHUNK - NRING + b, b)

        plsc.subcore_barrier()
        pltpu.sync_copy(acc_sh.at[pl.ds(s * RPT, RPT)],
                        out_hbm.at[c, pl.ds(s * RPT, RPT)])

    return spmm


_spmm_hid = _make_spmm(HID)
_spmm_out = _make_spmm(C_OUT)


def _dinv_of(deg_blk, i):
    deg = jnp.sum(deg_blk, axis=0)
    rowid = lax.broadcasted_iota(jnp.int32, (BLK,), 0) + i * BLK
    return jnp.where(rowid < N, lax.rsqrt(deg + 1.0), 0.0)


def _y1_body(deg_ref, x_ref, w1_ref, y_ref):
    dinv = _dinv_of(deg_ref[...], pl.program_id(0))
    xw = jnp.dot(x_ref[...], w1_ref[...], preferred_element_type=jnp.float32)
    y_ref[...] = (xw * dinv[:, None]).astype(jnp.bfloat16)


def _y2_body(deg_ref, s1a_ref, s1b_ref, y1_ref, b1_ref, w2_ref, y2_ref):
    dinv = _dinv_of(deg_ref[...], pl.program_id(0))
    agg = (s1a_ref[0].astype(jnp.float32) + s1b_ref[0].astype(jnp.float32)
           + y1_ref[...].astype(jnp.float32))
    h = jnp.maximum(agg * dinv[:, None] + b1_ref[...], 0.0)
    xw2 = jnp.dot(h, w2_ref[...], preferred_element_type=jnp.float32)
    y2_ref[...] = (xw2 * dinv[:, None]).astype(jnp.bfloat16)


def _out_body(deg_ref, s2a_ref, s2b_ref, y2_ref, b2_ref, out_ref):
    dinv = _dinv_of(deg_ref[...], pl.program_id(0))
    agg = (s2a_ref[0].astype(jnp.float32) + s2b_ref[0].astype(jnp.float32)
           + y2_ref[...].astype(jnp.float32))
    out_ref[...] = agg * dinv[:, None] + b2_ref[...]


def _deg_spec():
    return pl.BlockSpec((NW, BLK), lambda i: (0, i))


def _rows(d):
    return pl.BlockSpec((BLK, d), lambda i: (i, 0))


def _part_a(d):
    return pl.BlockSpec((1, BLK, d), lambda i: (0, i, 0))


def _part_b(d):
    return pl.BlockSpec((1, BLK, d), lambda i: (1, i, 0))


def _full(shape):
    return pl.BlockSpec(shape, lambda i: (0,) * len(shape))


def kernel(x, edge_index, W1, b1, W2, b2):
    src = edge_index[0]
    dst = edge_index[1]
    pad_idx = N + (jnp.arange(EPAD - E, dtype=jnp.int32) % (NPAD - N))
    src_p = jnp.concatenate([src, pad_idx]).reshape(NW, EPT)
    dst_p = jnp.concatenate([dst, pad_idx]).reshape(NW, NCHUNK, CW)
    zeros_hid = jnp.zeros((NPAD, HID), jnp.bfloat16)
    zeros_out = jnp.zeros((NPAD, C_OUT), jnp.bfloat16)

    x_p = jnp.concatenate(
        [x, jnp.zeros((NPAD - N, F_IN), jnp.float32)], axis=0)

    deg_parts = _deg_kernel(dst_p.reshape(NW, EPT))

    y1 = pl.pallas_call(
        _y1_body,
        grid=(GRID,),
        in_specs=[_deg_spec(), _rows(F_IN), _full((F_IN, HID))],
        out_specs=_rows(HID),
        out_shape=jax.ShapeDtypeStruct((NPAD, HID), jnp.bfloat16),
    )(deg_parts, x_p, W1)

    s1 = _spmm_hid(y1, src_p, dst_p, zeros_hid)

    y2 = pl.pallas_call(
        _y2_body,
        grid=(GRID,),
        in_specs=[_deg_spec(), _part_a(HID), _part_b(HID), _rows(HID),
                  _full((1, HID)), _full((HID, C_OUT))],
        out_specs=_rows(C_OUT),
        out_shape=jax.ShapeDtypeStruct((NPAD, C_OUT), jnp.bfloat16),
    )(deg_parts, s1, s1, y1, b1.reshape(1, HID), W2)

    s2 = _spmm_out(y2, src_p, dst_p, zeros_out)

    out = pl.pallas_call(
        _out_body,
        grid=(GRID,),
        in_specs=[_deg_spec(), _part_a(C_OUT), _part_b(C_OUT), _rows(C_OUT),
                  _full((1, C_OUT))],
        out_specs=_rows(C_OUT),
        out_shape=jax.ShapeDtypeStruct((NPAD, C_OUT), jnp.float32),
    )(deg_parts, s2, s2, y2, b2.reshape(1, C_OUT))

    return out[:N]


# speedup vs baseline: 1.0507x; 1.0053x over previous
import functools

import jax
import jax.numpy as jnp
from jax import lax
from jax.experimental import pallas as pl
from jax.experimental.pallas import tpu as pltpu
from jax.experimental.pallas import tpu_sc as plsc

N = 10000
E = 320000
F_IN = 128
HID = 128
C_OUT = 64

NPAD = 10240
NW = 32
CW = 128
NCHUNK = 80
EPT = NCHUNK * CW
EPAD = NW * EPT
RPT = NPAD // 16
BLK = 512
GRID = NPAD // BLK
NRING = 8
LAG = 7

_mesh = plsc.VectorSubcoreMesh(core_axis_name="c", subcore_axis_name="s")


@functools.partial(
    pl.kernel,
    out_type=jax.ShapeDtypeStruct((NW, NPAD), jnp.float32),
    mesh=_mesh,
    scratch_types=[
        pltpu.VMEM((EPT,), jnp.int32),
        pltpu.VMEM((NPAD,), jnp.float32),
    ],
    compiler_params=pltpu.CompilerParams(needs_layout_passes=False),
)
def _deg_kernel(dst_hbm, out_hbm, dst_v, deg_v):
    c = lax.axis_index("c")
    s = lax.axis_index("s")
    wid = s * 2 + c
    pltpu.sync_copy(dst_hbm.at[wid], dst_v)
    zeros = jnp.zeros((16,), jnp.float32)
    ones = jnp.ones((16,), jnp.float32)

    def zero_body(i, _):
        for u in range(4):
            deg_v[pl.ds((i * 4 + u) * 16, 16)] = zeros
        return 0

    lax.fori_loop(0, NPAD // 64, zero_body, 0)

    def body(i, _):
        idx = dst_v[pl.ds(i * 16, 16)]
        plsc.addupdate_scatter(deg_v, [idx], ones)
        return 0

    lax.fori_loop(0, EPT // 16, body, 0)
    pltpu.sync_copy(deg_v, out_hbm.at[wid])


def _make_spmm(d):
    @functools.partial(
        pl.kernel,
        out_type=jax.ShapeDtypeStruct((2, NPAD, d), jnp.bfloat16),
        mesh=_mesh,
        scratch_types=[
            pltpu.VMEM((EPT,), jnp.int32),
            pltpu.VMEM((NCHUNK, CW), jnp.int32),
            [pltpu.VMEM((CW, d), jnp.bfloat16) for _ in range(NRING)],
            pltpu.VMEM_SHARED((NPAD, d), jnp.bfloat16),
            [pltpu.SemaphoreType.DMA for _ in range(NRING)],
            [pltpu.SemaphoreType.DMA for _ in range(NRING)],
        ],
        compiler_params=pltpu.CompilerParams(use_tc_tiling_on_sc=False),
    )
    def spmm(y_hbm, src_hbm, dst_hbm, zeros_hbm, out_hbm,
             src_v, dst_v, bufs, acc_sh, gsems, ssems):
        c = lax.axis_index("c")
        s = lax.axis_index("s")
        wid = s * 2 + c
        pltpu.sync_copy(zeros_hbm.at[pl.ds(s * RPT, RPT)],
                        acc_sh.at[pl.ds(s * RPT, RPT)])
        pltpu.sync_copy(src_hbm.at[wid], src_v)
        pltpu.sync_copy(dst_hbm.at[wid], dst_v)

        def src_at(j):
            return y_hbm.at[src_v.at[pl.ds(j * CW, CW)]]

        def gather(j, b):
            pltpu.async_copy(src_at(j), bufs[b], gsems[b])

        def wait_gather(j, b):
            pltpu.make_async_copy(src_at(j), bufs[b], gsems[b]).wait()

        def scatter(j, b):
            pltpu.async_copy(bufs[b], acc_sh.at[dst_v.at[j]], ssems[b],
                             add=True)

        def wait_scatter(j, b):
            pltpu.make_async_copy(bufs[b], acc_sh.at[dst_v.at[j]],
                                  ssems[b]).wait()

        plsc.subcore_barrier()
        for b in range(LAG):
            gather(b, b)

        for jj in range(NRING):
            b = jj % NRING
            wait_gather(jj, b)
            scatter(jj, b)
            bg = (jj + LAG) % NRING
            if jj + LAG >= NRING:
                wait_scatter(jj + LAG - NRING, bg)
            gather(jj + LAG, bg)

        def body(i, _):
            for b in range(NRING):
                jj = i * NRING + b
                wait_gather(jj, b)
                scatter(jj, b)
                bg = (b + LAG) % NRING
                wait_scatter(jj + LAG - NRING, bg)
                gather(jj + LAG, bg)
            return 0

        lax.fori_loop(1, NCHUNK // NRING - 1, body, 0)

        for b in range(NRING):
            jj = NCHUNK - NRING + b
            wait_gather(jj, b)
            scatter(jj, b)
            if jj + LAG < NCHUNK:
                bg = (b + LAG) % NRING
                wait_scatter(jj + LAG - NRING, bg)
                gather(jj + LAG, bg)
        for b in range(NRING):
            wait_scatter(NCHUNK - NRING + b, b)

        plsc.subcore_barrier()
        pltpu.sync_copy(acc_sh.at[pl.ds(s * RPT, RPT)],
                        out_hbm.at[c, pl.ds(s * RPT, RPT)])

    return spmm


_spmm_hid = _make_spmm(HID)
_spmm_out = _make_spmm(C_OUT)


def _dinv_of(deg_blk, i):
    deg = jnp.sum(deg_blk, axis=0)
    rowid = lax.broadcasted_iota(jnp.int32, (BLK,), 0) + i * BLK
    return jnp.where(rowid < N, lax.rsqrt(deg + 1.0), 0.0)


def _y1_body(deg_ref, x_ref, w1_ref, y_ref):
    dinv = _dinv_of(deg_ref[...], pl.program_id(0))
    xw = jnp.dot(x_ref[...], w1_ref[...], preferred_element_type=jnp.float32)
    y_ref[...] = (xw * dinv[:, None]).astype(jnp.bfloat16)


def _y2_body(deg_ref, s1a_ref, s1b_ref, y1_ref, b1_ref, w2_ref, y2_ref):
    dinv = _dinv_of(deg_ref[...], pl.program_id(0))
    agg = (s1a_ref[0].astype(jnp.float32) + s1b_ref[0].astype(jnp.float32)
           + y1_ref[...].astype(jnp.float32))
    h = jnp.maximum(agg * dinv[:, None] + b1_ref[...], 0.0)
    xw2 = jnp.dot(h, w2_ref[...], preferred_element_type=jnp.float32)
    y2_ref[...] = (xw2 * dinv[:, None]).astype(jnp.bfloat16)


def _out_body(deg_ref, s2a_ref, s2b_ref, y2_ref, b2_ref, out_ref):
    dinv = _dinv_of(deg_ref[...], pl.program_id(0))
    agg = (s2a_ref[0].astype(jnp.float32) + s2b_ref[0].astype(jnp.float32)
           + y2_ref[...].astype(jnp.float32))
    out_ref[...] = agg * dinv[:, None] + b2_ref[...]


def _deg_spec():
    return pl.BlockSpec((NW, BLK), lambda i: (0, i))


def _rows(d):
    return pl.BlockSpec((BLK, d), lambda i: (i, 0))


def _part_a(d):
    return pl.BlockSpec((1, BLK, d), lambda i: (0, i, 0))


def _part_b(d):
    return pl.BlockSpec((1, BLK, d), lambda i: (1, i, 0))


def _full(shape):
    return pl.BlockSpec(shape, lambda i: (0,) * len(shape))


def kernel(x, edge_index, W1, b1, W2, b2):
    src = edge_index[0]
    dst = edge_index[1]
    pad_idx = N + (jnp.arange(EPAD - E, dtype=jnp.int32) % (NPAD - N))
    src_p = jnp.concatenate([src, pad_idx]).reshape(NW, EPT)
    dst_p = jnp.concatenate([dst, pad_idx]).reshape(NW, NCHUNK, CW)
    zeros_hid = jnp.zeros((NPAD, HID), jnp.bfloat16)
    zeros_out = jnp.zeros((NPAD, C_OUT), jnp.bfloat16)

    x_p = jnp.concatenate(
        [x, jnp.zeros((NPAD - N, F_IN), jnp.float32)], axis=0)

    deg_parts = _deg_kernel(dst_p.reshape(NW, EPT))

    y1 = pl.pallas_call(
        _y1_body,
        grid=(GRID,),
        in_specs=[_deg_spec(), _rows(F_IN), _full((F_IN, HID))],
        out_specs=_rows(HID),
        out_shape=jax.ShapeDtypeStruct((NPAD, HID), jnp.bfloat16),
    )(deg_parts, x_p, W1)

    s1 = _spmm_hid(y1, src_p, dst_p, zeros_hid)

    y2 = pl.pallas_call(
        _y2_body,
        grid=(GRID,),
        in_specs=[_deg_spec(), _part_a(HID), _part_b(HID), _rows(HID),
                  _full((1, HID)), _full((HID, C_OUT))],
        out_specs=_rows(C_OUT),
        out_shape=jax.ShapeDtypeStruct((NPAD, C_OUT), jnp.bfloat16),
    )(deg_parts, s1, s1, y1, b1.reshape(1, HID), W2)

    s2 = _spmm_out(y2, src_p, dst_p, zeros_out)

    out = pl.pallas_call(
        _out_body,
        grid=(GRID,),
        in_specs=[_deg_spec(), _part_a(C_OUT), _part_b(C_OUT), _rows(C_OUT),
                  _full((1, C_OUT))],
        out_specs=_rows(C_OUT),
        out_shape=jax.ShapeDtypeStruct((NPAD, C_OUT), jnp.float32),
    )(deg_parts, s2, s2, y2, b2.reshape(1, C_OUT))

    return out[:N]
